# baseline (speedup 1.0000x reference)
import functools

import jax
import jax.numpy as jnp
from jax import lax
from jax.experimental import pallas as pl
from jax.experimental.pallas import tpu as pltpu
from jax.experimental.pallas import tpu_sc as plsc

_B, _C, _L, _D, _K = 16, 256, 4096, 256, 128
_NB = 4

_NW = 32
_PW = (_B * _L) // _NW
_LT = 256


def _tc_body(x_ref, we_ref, be_ref, cb_ref, wd_ref, bd_ref,
             idx_ref, loss_ref, dcb_ref):
    first = pl.program_id(0) == 0

    @pl.when(first)
    def _init():
        dcb_ref[...] = jax.lax.dot_general(
            wd_ref[...], cb_ref[...],
            dimension_numbers=(((1,), (1,)), ((), ()))) + bd_ref[...]

    part = jnp.zeros((1, 1), jnp.float32)
    for bi in range(_NB):
        xb = x_ref[bi]
        zT = jnp.dot(we_ref[...], xb) + be_ref[...]
        scores = jnp.dot(cb_ref[...], zT)
        cb_sq = jnp.sum(cb_ref[...] * cb_ref[...], axis=1, keepdims=True)
        e = cb_sq - 2.0 * scores
        mine = jnp.min(e, axis=0)
        iota_k = jax.lax.broadcasted_iota(jnp.int32, (_K, _L), 0)
        idx = jnp.min(jnp.where(e == mine[None, :], iota_k, _K), axis=0)
        idx_ref[bi, 0, :] = idx
        part = part + (jnp.sum(zT * zT) + jnp.sum(mine)).reshape(1, 1)

    @pl.when(first)
    def _set():
        loss_ref[...] = part

    @pl.when(jnp.logical_not(first))
    def _acc():
        loss_ref[...] += part


def _tc_call(x, W_enc, b_enc, codebook, W_dec, b_dec):
    return pl.pallas_call(
        _tc_body,
        grid=(_B // _NB,),
        in_specs=[
            pl.BlockSpec((_NB, _C, _L), lambda i: (i, 0, 0)),
            pl.BlockSpec((_D, _C), lambda i: (0, 0)),
            pl.BlockSpec((_D, 1), lambda i: (0, 0)),
            pl.BlockSpec((_K, _D), lambda i: (0, 0)),
            pl.BlockSpec((_C, _D), lambda i: (0, 0)),
            pl.BlockSpec((_C, 1), lambda i: (0, 0)),
        ],
        out_specs=[
            pl.BlockSpec((_NB, 1, _L), lambda i: (i, 0, 0)),
            pl.BlockSpec((1, 1), lambda i: (0, 0)),
            pl.BlockSpec((_C, _K), lambda i: (0, 0)),
        ],
        out_shape=[
            jax.ShapeDtypeStruct((_B, 1, _L), jnp.int32),
            jax.ShapeDtypeStruct((1, 1), jnp.float32),
            jax.ShapeDtypeStruct((_C, _K), jnp.float32),
        ],
    )(x, W_enc, b_enc.reshape(_D, 1), codebook, W_dec, b_dec.reshape(_C, 1))


@functools.partial(
    pl.kernel,
    mesh=plsc.VectorSubcoreMesh(core_axis_name="c", subcore_axis_name="s"),
    out_type=jax.ShapeDtypeStruct((_B, _C, _L), jnp.float32),
    scratch_types=[
        pltpu.VMEM((_PW,), jnp.int32),
        pltpu.VMEM((_C * _K,), jnp.float32),
        pltpu.VMEM((_C, _LT), jnp.float32),
    ],
    compiler_params=pltpu.CompilerParams(needs_layout_passes=False),
)
def _sc_gather(dcb_hbm, idxf_hbm, out_hbm, idx_v, tab_v, buf_v):
    w = lax.axis_index("s") * 2 + lax.axis_index("c")
    b = w // 2
    l0 = (w % 2) * _PW
    pltpu.sync_copy(idxf_hbm.at[pl.ds(w * _PW, _PW)], idx_v)
    pltpu.sync_copy(dcb_hbm, tab_v)
    for sub in range(_PW // _LT):
        def cbody(c, carry, sub=sub):
            base_t = c * _K
            for v in range(_LT // 16):
                i16 = idx_v[pl.ds(sub * _LT + v * 16, 16)]
                vals = plsc.load_gather(tab_v, [base_t + i16])
                buf_v[c, pl.ds(v * 16, 16)] = vals
            return carry
        lax.fori_loop(0, _C, cbody, 0)
        pltpu.sync_copy(buf_v, out_hbm.at[b, :, pl.ds(l0 + sub * _LT, _LT)])


def kernel(x, W_enc, b_enc, codebook, W_dec, b_dec):
    idx3, loss_sum, dcb = _tc_call(x, W_enc, b_enc, codebook, W_dec, b_dec)
    out = _sc_gather(dcb.reshape(_C * _K), idx3.reshape(_B * _L))
    indices = idx3.reshape(_B, _L)
    commit_loss = (loss_sum[0, 0] / (_B * _L * _D)).astype(jnp.float32)
    return (out, indices, commit_loss)


# speedup vs baseline: 7.2850x; 7.2850x over previous
import jax
import jax.numpy as jnp
from jax.experimental import pallas as pl
from jax.experimental.pallas import tpu as pltpu

_B, _C, _L, _D, _K = 16, 256, 4096, 256, 128
_NB = 2


def _vq_body(x_ref, we_ref, be_ref, cb_ref, wd_ref, bd_ref,
             out_ref, idx_ref, loss_ref, dcb_ref):
    first = pl.program_id(0) == 0

    @pl.when(first)
    def _init():
        dcb = jax.lax.dot_general(
            wd_ref[...], cb_ref[...],
            dimension_numbers=(((1,), (1,)), ((), ()))) + bd_ref[...]
        dcb_ref[...] = dcb.astype(jnp.bfloat16)

    for _bi in range(_NB):
        _vq_one(x_ref, we_ref, be_ref, cb_ref, out_ref, idx_ref, loss_ref,
                dcb_ref, first & (_bi == 0), _bi)


def _vq_one(x_ref, we_ref, be_ref, cb_ref, out_ref, idx_ref, loss_ref,
            dcb_ref, first, bi):
    xb = x_ref[bi]
    zT = jnp.dot(we_ref[...], xb) + be_ref[...]
    scores = jnp.dot(cb_ref[...], zT)
    cb_sq = jnp.sum(cb_ref[...] * cb_ref[...], axis=1, keepdims=True)
    e = cb_sq - 2.0 * scores

    mine = jnp.min(e, axis=0)
    iota_k = jax.lax.broadcasted_iota(jnp.int32, (_K, _L), 0)
    idx = jnp.min(jnp.where(e == mine[None, :], iota_k, _K), axis=0)
    idx_ref[bi, 0, :] = idx

    onehot = (iota_k == idx[None, :]).astype(jnp.bfloat16)
    out_ref[bi] = jax.lax.dot_general(
        dcb_ref[...], onehot, dimension_numbers=(((1,), (0,)), ((), ())),
        preferred_element_type=jnp.float32)

    part = (jnp.sum(zT * zT) + jnp.sum(mine)).reshape(1, 1)

    @pl.when(first)
    def _set():
        loss_ref[...] = part

    @pl.when(jnp.logical_not(first))
    def _acc():
        loss_ref[...] += part


def kernel(x, W_enc, b_enc, codebook, W_dec, b_dec):
    out, idx3, loss_sum = pl.pallas_call(
        _vq_body,
        grid=(_B // _NB,),
        in_specs=[
            pl.BlockSpec((_NB, _C, _L), lambda i: (i, 0, 0)),
            pl.BlockSpec((_D, _C), lambda i: (0, 0)),
            pl.BlockSpec((_D, 1), lambda i: (0, 0)),
            pl.BlockSpec((_K, _D), lambda i: (0, 0)),
            pl.BlockSpec((_C, _D), lambda i: (0, 0)),
            pl.BlockSpec((_C, 1), lambda i: (0, 0)),
        ],
        out_specs=[
            pl.BlockSpec((_NB, _C, _L), lambda i: (i, 0, 0)),
            pl.BlockSpec((_NB, 1, _L), lambda i: (i, 0, 0)),
            pl.BlockSpec((1, 1), lambda i: (0, 0)),
        ],
        out_shape=[
            jax.ShapeDtypeStruct((_B, _C, _L), jnp.float32),
            jax.ShapeDtypeStruct((_B, 1, _L), jnp.int32),
            jax.ShapeDtypeStruct((1, 1), jnp.float32),
        ],
        scratch_shapes=[pltpu.VMEM((_C, _K), jnp.bfloat16)],
    )(x, W_enc, b_enc.reshape(_D, 1), codebook, W_dec, b_dec.reshape(_C, 1))
    indices = idx3.reshape(_B, _L)
    commit_loss = (loss_sum[0, 0] / (_B * _L * _D)).astype(jnp.float32)
    return (out, indices, commit_loss)
